# Initial kernel scaffold; baseline (speedup 1.0000x reference)
import jax
import jax.numpy as jnp
from jax.experimental import pallas as pl


def kernel(x0, W1, b1, g1, be1, W2, b2, g2, be2, cw1, cb1, cw2, cb2):
    raise NotImplementedError("write your pallas kernel here")


# speedup vs baseline: 37.3187x; 37.3187x over previous
import functools

import jax
import jax.numpy as jnp
import numpy as np
from jax.experimental import pallas as pl

NP0, NP1 = 20, 20
NPREF = NP0 * NP1
NDIM = 3
B, S = 4, 4096
END_RATE = 0.05
INHIB = 0.1

_CHUNK = 2048
_HIGHEST = jax.lax.Precision.HIGHEST


def _build_conv_affine(cw1, cb1, cw2, cb2):
    C2 = cw2.sum(axis=0)
    K = jnp.zeros((5, 5), jnp.float32)
    for ai in range(3):
        for aj in range(3):
            K = K.at[ai:ai + 3, aj:aj + 3].add(
                jnp.einsum('i,ixy->xy', cw1[:, 0, ai, aj], C2))
    K = K / 10.0
    c = (cb2.sum() + (cb1 * cw2.sum(axis=(0, 2, 3))).sum()) / 10.0
    yy, xx = np.meshgrid(np.arange(NP0), np.arange(NP1), indexing='ij')
    q = (NP1 * yy + xx).reshape(-1)
    M = jnp.zeros((NPREF, NPREF), jnp.float32)
    for dy in range(-2, 3):
        for dx in range(-2, 3):
            p = (NP1 * ((yy + dy) % NP0) + ((xx + dx) % NP1)).reshape(-1)
            M = M.at[q, p].add(K[dy + 2, dx + 2])
    return M, c.reshape(1, 1)


def _net_kernel(x_ref, w1_ref, b1_ref, g1_ref, be1_ref, w2_ref, b2_ref,
                g2_ref, be2_ref, m_ref, c_ref, wt_ref, xm2_ref):
    x = x_ref[0]
    w1 = w1_ref[...]
    h = (w1[:, 0:1] * x[0:1, :] + w1[:, 1:2] * x[1:2, :]
         + w1[:, 2:3] * x[2:3, :] + b1_ref[...])
    h = jnp.maximum(h, 0.0)
    mu = jnp.mean(h, axis=0, keepdims=True)
    var = jnp.mean((h - mu) ** 2, axis=0, keepdims=True)
    h = (h - mu) * jax.lax.rsqrt(var + 1e-5) * g1_ref[...] + be1_ref[...]

    h = jax.lax.dot_general(w2_ref[...], h, (((1,), (0,)), ((), ())),
                            precision=_HIGHEST,
                            preferred_element_type=jnp.float32) + b2_ref[...]
    h = jnp.maximum(h, 0.0)
    mu = jnp.mean(h, axis=0, keepdims=True)
    var = jnp.mean((h - mu) ** 2, axis=0, keepdims=True)
    h = (h - mu) * jax.lax.rsqrt(var + 1e-5) * g2_ref[...] + be2_ref[...]

    w = jax.lax.dot_general(m_ref[...], h, (((1,), (0,)), ((), ())),
                            precision=_HIGHEST,
                            preferred_element_type=jnp.float32) + c_ref[0, 0]
    wt_ref[0] = jax.nn.sigmoid(w)

    xm = (x[0:1, :] + x[1:2, :] + x[2:3, :]) * (1.0 / 3.0)
    xm2_ref[0] = xm * xm


def _select_kernel(wt_ref, xm2_ref, out_ref):
    iota = jax.lax.broadcasted_iota(jnp.int32, (NPREF, 1), 0)

    def body(state):
        t, resid, selmask, done = state
        new_resid, new_sel, new_done = [], [], []
        for b in range(B):
            rb = resid[b]
            rr = rb * rb * xm2_ref[b]
            wb = wt_ref[b]
            act2 = jnp.sum(wb * wb * rr, axis=1, keepdims=True)
            act2 = jnp.where(selmask[b] > 0.5, 0.0, act2)
            mx = jnp.max(act2)
            p = jnp.min(jnp.where(act2 == mx, iota, NPREF))
            onehot = (iota == p).astype(jnp.float32)
            already = jnp.max(onehot * selmask[b])
            active = 1.0 - done[b]
            wp = wt_ref[b, pl.ds(p, 1), :]
            gate = active * (1.0 - already)
            rb2 = jnp.maximum(rb - wp * gate, 0.0)
            new_resid.append(rb2)
            new_sel.append(jnp.maximum(selmask[b], onehot * active))
            new_done.append(jnp.maximum(
                done[b],
                jnp.where(jnp.mean(rb2) < END_RATE, 1.0, 0.0)))
        return (t + 1, tuple(new_resid), tuple(new_sel), tuple(new_done))

    def cond(state):
        t, _, _, done = state
        n_done = functools.reduce(jnp.add, done)
        return jnp.logical_and(t < NPREF, n_done < B - 0.5)

    state0 = (
        jnp.int32(0),
        tuple(jnp.ones((1, S), jnp.float32) for _ in range(B)),
        tuple(jnp.zeros((NPREF, 1), jnp.float32) for _ in range(B)),
        tuple(jnp.zeros((), jnp.float32) for _ in range(B)),
    )
    _, _, selmask, _ = jax.lax.while_loop(cond, body, state0)
    for b in range(B):
        scale = jnp.where(selmask[b] > 0.5, 1.0, INHIB)
        out_ref[b] = wt_ref[b] * scale


def kernel(x0, W1, b1, g1, be1, W2, b2, g2, be2, cw1, cb1, cw2, cb2):
    M, c = _build_conv_affine(cw1, cb1, cw2, cb2)
    x0_t = jnp.transpose(x0, (0, 2, 1))
    col = lambda v: v.reshape(NPREF, 1)

    n_chunks = S // _CHUNK
    grid = (B * n_chunks,)
    full = lambda shape: pl.BlockSpec(shape, lambda i: (0,) * len(shape))
    wt, xm2 = pl.pallas_call(
        _net_kernel,
        grid=grid,
        in_specs=[
            pl.BlockSpec((1, NDIM, _CHUNK),
                         lambda i: (i // n_chunks, 0, i % n_chunks)),
            full((NPREF, NDIM)), full((NPREF, 1)), full((NPREF, 1)),
            full((NPREF, 1)), full((NPREF, NPREF)), full((NPREF, 1)),
            full((NPREF, 1)), full((NPREF, 1)), full((NPREF, NPREF)),
            full((1, 1)),
        ],
        out_specs=[
            pl.BlockSpec((1, NPREF, _CHUNK),
                         lambda i: (i // n_chunks, 0, i % n_chunks)),
            pl.BlockSpec((1, 1, _CHUNK),
                         lambda i: (i // n_chunks, 0, i % n_chunks)),
        ],
        out_shape=[
            jax.ShapeDtypeStruct((B, NPREF, S), jnp.float32),
            jax.ShapeDtypeStruct((B, 1, S), jnp.float32),
        ],
        name="pd_net",
    )(x0_t, W1, col(b1), col(g1), col(be1), W2, col(b2), col(g2), col(be2),
      M, c)

    out = pl.pallas_call(
        _select_kernel,
        out_shape=jax.ShapeDtypeStruct((B, NPREF, S), jnp.float32),
        input_output_aliases={0: 0},
        name="pd_select",
    )(wt, xm2)
    return out


# speedup vs baseline: 202.9891x; 5.4393x over previous
import functools

import jax
import jax.numpy as jnp
import numpy as np
from jax.experimental import pallas as pl

NP0, NP1 = 20, 20
NPREF = NP0 * NP1
NDIM = 3
B, S = 4, 4096
END_RATE = 0.05
INHIB = 0.1

_CHUNK = 2048
_HIGHEST = jax.lax.Precision.HIGHEST


def _build_conv_affine(cw1, cb1, cw2, cb2):
    C2 = cw2.sum(axis=0)
    K = jnp.zeros((5, 5), jnp.float32)
    for ai in range(3):
        for aj in range(3):
            K = K.at[ai:ai + 3, aj:aj + 3].add(
                jnp.einsum('i,ixy->xy', cw1[:, 0, ai, aj], C2))
    K = K / 10.0
    c = (cb2.sum() + (cb1 * cw2.sum(axis=(0, 2, 3))).sum()) / 10.0
    K20 = jnp.roll(jnp.pad(K, ((0, NP0 - 5), (0, NP1 - 5))), (-2, -2), (0, 1))
    ar = np.arange(NP0)
    ry = ((ar[None, :] - ar[:, None]) % NP0)
    basis = (ry[None, :, :] == ar[:, None, None]).astype(np.float32)
    cx = jnp.einsum('uv,vcd->ucd', K20, basis)
    M = jnp.einsum('uab,ucd->acbd', basis, cx).reshape(NPREF, NPREF)
    return M, c.reshape(1, 1)


def _net_kernel(x_ref, w1_ref, b1_ref, g1_ref, be1_ref, w2_ref, b2_ref,
                g2_ref, be2_ref, m_ref, c_ref, wt_ref, xm2_ref):
    x = x_ref[0]
    w1 = w1_ref[...]
    h = (w1[:, 0:1] * x[0:1, :] + w1[:, 1:2] * x[1:2, :]
         + w1[:, 2:3] * x[2:3, :] + b1_ref[...])
    h = jnp.maximum(h, 0.0)
    mu = jnp.mean(h, axis=0, keepdims=True)
    var = jnp.mean((h - mu) ** 2, axis=0, keepdims=True)
    h = (h - mu) * jax.lax.rsqrt(var + 1e-5) * g1_ref[...] + be1_ref[...]

    h = jax.lax.dot_general(w2_ref[...], h, (((1,), (0,)), ((), ())),
                            precision=_HIGHEST,
                            preferred_element_type=jnp.float32) + b2_ref[...]
    h = jnp.maximum(h, 0.0)
    mu = jnp.mean(h, axis=0, keepdims=True)
    var = jnp.mean((h - mu) ** 2, axis=0, keepdims=True)
    h = (h - mu) * jax.lax.rsqrt(var + 1e-5) * g2_ref[...] + be2_ref[...]

    w = jax.lax.dot_general(m_ref[...], h, (((1,), (0,)), ((), ())),
                            precision=_HIGHEST,
                            preferred_element_type=jnp.float32) + c_ref[0, 0]
    wt_ref[0] = jax.nn.sigmoid(w)

    xm = (x[0:1, :] + x[1:2, :] + x[2:3, :]) * (1.0 / 3.0)
    xm2_ref[0] = xm * xm


def _select_kernel(wt_ref, xm2_ref, out_ref):
    iota = jax.lax.broadcasted_iota(jnp.int32, (NPREF, 1), 0)

    def body(state):
        t, resid, selmask, done = state
        new_resid, new_sel, new_done = [], [], []
        for b in range(B):
            rb = resid[b]
            rr = rb * rb * xm2_ref[b]
            wb = wt_ref[b]
            act2 = jnp.sum(wb * wb * rr, axis=1, keepdims=True)
            act2 = jnp.where(selmask[b] > 0.5, 0.0, act2)
            mx = jnp.max(act2)
            p = jnp.min(jnp.where(act2 == mx, iota, NPREF))
            onehot = (iota == p).astype(jnp.float32)
            already = jnp.max(onehot * selmask[b])
            active = 1.0 - done[b]
            wp = wt_ref[b, pl.ds(p, 1), :]
            gate = active * (1.0 - already)
            rb2 = jnp.maximum(rb - wp * gate, 0.0)
            new_resid.append(rb2)
            new_sel.append(jnp.maximum(selmask[b], onehot * active))
            new_done.append(jnp.maximum(
                done[b],
                jnp.where(jnp.mean(rb2) < END_RATE, 1.0, 0.0)))
        return (t + 1, tuple(new_resid), tuple(new_sel), tuple(new_done))

    def cond(state):
        t, _, _, done = state
        n_done = functools.reduce(jnp.add, done)
        return jnp.logical_and(t < NPREF, n_done < B - 0.5)

    state0 = (
        jnp.int32(0),
        tuple(jnp.ones((1, S), jnp.float32) for _ in range(B)),
        tuple(jnp.zeros((NPREF, 1), jnp.float32) for _ in range(B)),
        tuple(jnp.zeros((), jnp.float32) for _ in range(B)),
    )
    _, _, selmask, _ = jax.lax.while_loop(cond, body, state0)
    for b in range(B):
        scale = jnp.where(selmask[b] > 0.5, 1.0, INHIB)
        out_ref[b] = wt_ref[b] * scale


def kernel(x0, W1, b1, g1, be1, W2, b2, g2, be2, cw1, cb1, cw2, cb2):
    M, c = _build_conv_affine(cw1, cb1, cw2, cb2)
    x0_t = jnp.transpose(x0, (0, 2, 1))
    col = lambda v: v.reshape(NPREF, 1)

    n_chunks = S // _CHUNK
    grid = (B * n_chunks,)
    full = lambda shape: pl.BlockSpec(shape, lambda i: (0,) * len(shape))
    wt, xm2 = pl.pallas_call(
        _net_kernel,
        grid=grid,
        in_specs=[
            pl.BlockSpec((1, NDIM, _CHUNK),
                         lambda i: (i // n_chunks, 0, i % n_chunks)),
            full((NPREF, NDIM)), full((NPREF, 1)), full((NPREF, 1)),
            full((NPREF, 1)), full((NPREF, NPREF)), full((NPREF, 1)),
            full((NPREF, 1)), full((NPREF, 1)), full((NPREF, NPREF)),
            full((1, 1)),
        ],
        out_specs=[
            pl.BlockSpec((1, NPREF, _CHUNK),
                         lambda i: (i // n_chunks, 0, i % n_chunks)),
            pl.BlockSpec((1, 1, _CHUNK),
                         lambda i: (i // n_chunks, 0, i % n_chunks)),
        ],
        out_shape=[
            jax.ShapeDtypeStruct((B, NPREF, S), jnp.float32),
            jax.ShapeDtypeStruct((B, 1, S), jnp.float32),
        ],
        name="pd_net",
    )(x0_t, W1, col(b1), col(g1), col(be1), W2, col(b2), col(g2), col(be2),
      M, c)

    out = pl.pallas_call(
        _select_kernel,
        out_shape=jax.ShapeDtypeStruct((B, NPREF, S), jnp.float32),
        input_output_aliases={0: 0},
        name="pd_select",
    )(wt, xm2)
    return out


# speedup vs baseline: 260.0062x; 1.2809x over previous
import functools

import jax
import jax.numpy as jnp
import numpy as np
from jax.experimental import pallas as pl

NP0, NP1 = 20, 20
NPREF = NP0 * NP1
NDIM = 3
B, S = 4, 4096
END_RATE = 0.05
INHIB = 0.1

_CHUNK = 2048
_HIGHEST = jax.lax.Precision.HIGHEST


def _build_conv_affine(cw1, cb1, cw2, cb2):
    C2 = cw2.sum(axis=0)
    K = jnp.zeros((5, 5), jnp.float32)
    for ai in range(3):
        for aj in range(3):
            K = K.at[ai:ai + 3, aj:aj + 3].add(
                jnp.einsum('i,ixy->xy', cw1[:, 0, ai, aj], C2))
    K = K / 10.0
    c = (cb2.sum() + (cb1 * cw2.sum(axis=(0, 2, 3))).sum()) / 10.0
    K20 = jnp.roll(jnp.pad(K, ((0, NP0 - 5), (0, NP1 - 5))), (-2, -2), (0, 1))
    ar = np.arange(NP0)
    ry = ((ar[None, :] - ar[:, None]) % NP0)
    basis = (ry[None, :, :] == ar[:, None, None]).astype(np.float32)
    cx = jnp.einsum('uv,vcd->ucd', K20, basis)
    M = jnp.einsum('uab,ucd->acbd', basis, cx).reshape(NPREF, NPREF)
    return M, c.reshape(1, 1)


def _split_bf16(a):
    hi = a.astype(jnp.bfloat16)
    lo = (a - hi.astype(jnp.float32)).astype(jnp.bfloat16)
    return hi, lo


def _dot3(a_hi, a_lo, b):
    b_hi, b_lo = _split_bf16(b)
    dims = (((1,), (0,)), ((), ()))
    dot = functools.partial(jax.lax.dot_general, dimension_numbers=dims,
                            preferred_element_type=jnp.float32)
    return dot(a_hi, b_hi) + (dot(a_hi, b_lo) + dot(a_lo, b_hi))


def _net_kernel(x_ref, w1_ref, b1_ref, g1_ref, be1_ref, w2h_ref, w2l_ref,
                b2_ref, g2_ref, be2_ref, mh_ref, ml_ref, c_ref,
                wt_ref, xm2_ref):
    x = x_ref[0]
    w1 = w1_ref[...]
    h = (w1[:, 0:1] * x[0:1, :] + w1[:, 1:2] * x[1:2, :]
         + w1[:, 2:3] * x[2:3, :] + b1_ref[...])
    h = jnp.maximum(h, 0.0)
    mu = jnp.mean(h, axis=0, keepdims=True)
    var = jnp.mean((h - mu) ** 2, axis=0, keepdims=True)
    h = (h - mu) * jax.lax.rsqrt(var + 1e-5) * g1_ref[...] + be1_ref[...]

    h = _dot3(w2h_ref[...], w2l_ref[...], h) + b2_ref[...]
    h = jnp.maximum(h, 0.0)
    mu = jnp.mean(h, axis=0, keepdims=True)
    var = jnp.mean((h - mu) ** 2, axis=0, keepdims=True)
    h = (h - mu) * jax.lax.rsqrt(var + 1e-5) * g2_ref[...] + be2_ref[...]

    w = _dot3(mh_ref[...], ml_ref[...], h) + c_ref[0, 0]
    wt_ref[0] = jax.nn.sigmoid(w)

    xm = (x[0:1, :] + x[1:2, :] + x[2:3, :]) * (1.0 / 3.0)
    xm2_ref[0] = xm * xm


def _select_kernel(wt_ref, xm2_ref, out_ref):
    iota = jax.lax.broadcasted_iota(jnp.int32, (NPREF, 1), 0)

    def body(state):
        t, resid, selmask, done = state
        new_resid, new_sel, new_done = [], [], []
        for b in range(B):
            rb = resid[b]
            rr = rb * rb * xm2_ref[b]
            wb = wt_ref[b]
            act2 = jnp.sum(wb * wb * rr, axis=1, keepdims=True)
            act2 = jnp.where(selmask[b] > 0.5, 0.0, act2)
            mx = jnp.max(act2)
            p = jnp.min(jnp.where(act2 == mx, iota, NPREF))
            onehot = (iota == p).astype(jnp.float32)
            already = jnp.max(onehot * selmask[b])
            active = 1.0 - done[b]
            wp = wt_ref[b, pl.ds(p, 1), :]
            gate = active * (1.0 - already)
            rb2 = jnp.maximum(rb - wp * gate, 0.0)
            new_resid.append(rb2)
            new_sel.append(jnp.maximum(selmask[b], onehot * active))
            new_done.append(jnp.maximum(
                done[b],
                jnp.where(jnp.mean(rb2) < END_RATE, 1.0, 0.0)))
        return (t + 1, tuple(new_resid), tuple(new_sel), tuple(new_done))

    def cond(state):
        t, _, _, done = state
        n_done = functools.reduce(jnp.add, done)
        return jnp.logical_and(t < NPREF, n_done < B - 0.5)

    state0 = (
        jnp.int32(0),
        tuple(jnp.ones((1, S), jnp.float32) for _ in range(B)),
        tuple(jnp.zeros((NPREF, 1), jnp.float32) for _ in range(B)),
        tuple(jnp.zeros((), jnp.float32) for _ in range(B)),
    )
    _, _, selmask, _ = jax.lax.while_loop(cond, body, state0)
    for b in range(B):
        scale = jnp.where(selmask[b] > 0.5, 1.0, INHIB)
        out_ref[b] = wt_ref[b] * scale


def kernel(x0, W1, b1, g1, be1, W2, b2, g2, be2, cw1, cb1, cw2, cb2):
    M, c = _build_conv_affine(cw1, cb1, cw2, cb2)
    x0_t = jnp.transpose(x0, (0, 2, 1))
    col = lambda v: v.reshape(NPREF, 1)
    w2h = W2.astype(jnp.bfloat16)
    w2l = (W2 - w2h.astype(jnp.float32)).astype(jnp.bfloat16)
    mh = M.astype(jnp.bfloat16)
    ml = (M - mh.astype(jnp.float32)).astype(jnp.bfloat16)

    n_chunks = S // _CHUNK
    grid = (B * n_chunks,)
    full = lambda shape: pl.BlockSpec(shape, lambda i: (0,) * len(shape))
    wt, xm2 = pl.pallas_call(
        _net_kernel,
        grid=grid,
        in_specs=[
            pl.BlockSpec((1, NDIM, _CHUNK),
                         lambda i: (i // n_chunks, 0, i % n_chunks)),
            full((NPREF, NDIM)), full((NPREF, 1)), full((NPREF, 1)),
            full((NPREF, 1)), full((NPREF, NPREF)), full((NPREF, NPREF)),
            full((NPREF, 1)), full((NPREF, 1)), full((NPREF, 1)),
            full((NPREF, NPREF)), full((NPREF, NPREF)), full((1, 1)),
        ],
        out_specs=[
            pl.BlockSpec((1, NPREF, _CHUNK),
                         lambda i: (i // n_chunks, 0, i % n_chunks)),
            pl.BlockSpec((1, 1, _CHUNK),
                         lambda i: (i // n_chunks, 0, i % n_chunks)),
        ],
        out_shape=[
            jax.ShapeDtypeStruct((B, NPREF, S), jnp.float32),
            jax.ShapeDtypeStruct((B, 1, S), jnp.float32),
        ],
        name="pd_net",
    )(x0_t, W1, col(b1), col(g1), col(be1), w2h, w2l, col(b2), col(g2),
      col(be2), mh, ml, c)

    out = pl.pallas_call(
        _select_kernel,
        out_shape=jax.ShapeDtypeStruct((B, NPREF, S), jnp.float32),
        input_output_aliases={0: 0},
        name="pd_select",
    )(wt, xm2)
    return out
